# baseline (speedup 1.0000x reference)
import functools
import jax
import jax.numpy as jnp
from jax import lax
from jax.experimental import pallas as pl
from jax.experimental.pallas import tpu as pltpu
from jax.experimental.pallas import tpu_sc as plsc

NC = 2
NS = 16
LANES = 16


def kernel(x, pos_table):
    B, S, D = x.shape
    NW = NC * NS
    rows_per_w = S // NW
    R = 32
    n_chunks = rows_per_w // R
    nc2 = n_chunks // 2

    mesh = plsc.VectorSubcoreMesh(
        core_axis_name="c", subcore_axis_name="s", num_cores=NC, num_subcores=NS
    )

    @functools.partial(
        pl.kernel,
        mesh=mesh,
        out_type=jax.ShapeDtypeStruct((B, S, D), x.dtype),
        scratch_types=[
            pltpu.VMEM((2, R, D), jnp.float32),
            pltpu.VMEM((2, R, D), jnp.float32),
            pltpu.SemaphoreType.DMA((2,)),
            pltpu.SemaphoreType.DMA((2,)),
            pltpu.SemaphoreType.DMA((2,)),
        ],
    )
    def sc_add(x_hbm, pos_hbm, out_hbm, pbuf, xbuf, psem, insem, outsem):
        wid = lax.axis_index("s") * NC + lax.axis_index("c")
        base = wid * rows_per_w

        def pos_copy(c, par):
            return pltpu.make_async_copy(
                pos_hbm.at[pl.ds(base + c * R, R), :], pbuf.at[par], psem.at[par]
            )

        def x_copy(c, b, slot):
            return pltpu.make_async_copy(
                x_hbm.at[b, pl.ds(base + c * R, R), :],
                xbuf.at[slot],
                insem.at[slot],
            )

        def out_copy(c, b, slot):
            return pltpu.make_async_copy(
                xbuf.at[slot],
                out_hbm.at[b, pl.ds(base + c * R, R), :],
                outsem.at[slot],
            )

        pos_copy(0, 0).start()
        x_copy(0, 0, 0).start()

        def element(c, c2, par):
            pos_copy(c, par).wait()
            if par == 0:
                pos_copy(c + 1, 1).start()
            else:
                @pl.when(c2 < nc2 - 1)
                def _():
                    pos_copy(c + 1, 0).start()

            for b in range(B):
                p = b % 2
                x_copy(c, b, p).wait()

                def row_body(r, rcarry):
                    for j in range(D // LANES):
                        sl = pl.ds(j * LANES, LANES)
                        xbuf[p, r, sl] = xbuf[p, r, sl] + pbuf[par, r, sl]
                    return rcarry

                lax.fori_loop(0, R, row_body, 0)
                out_copy(c, b, p).start()

                if b == 0:
                    if par == 0:
                        @pl.when(c2 > 0)
                        def _():
                            out_copy(c - 1, B - 1, 1).wait()
                    else:
                        out_copy(c - 1, B - 1, 1).wait()
                    x_copy(c, 1, 1).start()
                elif b < B - 1:
                    out_copy(c, b - 1, 1 - p).wait()
                    x_copy(c, b + 1, 1 - p).start()
                else:
                    out_copy(c, b - 1, 1 - p).wait()
                    if par == 0:
                        x_copy(c + 1, 0, 0).start()
                    else:
                        @pl.when(c2 < nc2 - 1)
                        def _():
                            x_copy(c + 1, 0, 0).start()

        def pair_body(c2, carry):
            element(2 * c2, c2, 0)
            element(2 * c2 + 1, c2, 1)
            return carry

        lax.fori_loop(0, nc2, pair_body, 0)

        out_copy(n_chunks - 1, B - 1, 1).wait()

    return sc_add(x, pos_table)


# speedup vs baseline: 3.3317x; 3.3317x over previous
import jax
import jax.numpy as jnp
from jax.experimental import pallas as pl
from jax.experimental.pallas import tpu as pltpu


def _add_kernel(x_ref, pos_ref, out_ref):
    out_ref[...] = x_ref[...] + pos_ref[...][None, :, :]


def kernel(x, pos_table):
    B, S, D = x.shape
    S_BLK = 2048
    grid = (S // S_BLK, B)
    return pl.pallas_call(
        _add_kernel,
        grid=grid,
        in_specs=[
            pl.BlockSpec((1, S_BLK, D), lambda s, b: (b, s, 0)),
            pl.BlockSpec((S_BLK, D), lambda s, b: (s, 0)),
        ],
        out_specs=pl.BlockSpec((1, S_BLK, D), lambda s, b: (b, s, 0)),
        out_shape=jax.ShapeDtypeStruct((B, S, D), x.dtype),
        compiler_params=pltpu.CompilerParams(
            dimension_semantics=("parallel", "arbitrary")
        ),
    )(x, pos_table)
